# Initial kernel scaffold; baseline (speedup 1.0000x reference)
import jax
import jax.numpy as jnp
from jax.experimental import pallas as pl


def kernel(hidden_states, cu_seqlens, Wq, Wk, temperature, boundary_bias):
    raise NotImplementedError("write your pallas kernel here")


# speedup vs baseline: 1.2618x; 1.2618x over previous
import functools

import jax
import jax.numpy as jnp
from jax.experimental import pallas as pl
from jax.experimental.pallas import tpu as pltpu


def _routing_body(cu_ref, tb_ref, h_ref, wq_ref, wk_ref,
                  prob_ref, mask_ref, sel_ref, carry_ref, *, block_rows):
    i = pl.program_id(0)
    h = h_ref[...]
    q = jax.lax.dot_general(h, wq_ref[...], (((1,), (1,)), ((), ())),
                            preferred_element_type=jnp.float32)
    k = jax.lax.dot_general(h, wk_ref[...], (((1,), (1,)), ((), ())),
                            preferred_element_type=jnp.float32)
    qn = q / jnp.maximum(jnp.sqrt(jnp.sum(q * q, axis=1, keepdims=True)), 1e-12)
    kn = k / jnp.maximum(jnp.sqrt(jnp.sum(k * k, axis=1, keepdims=True)), 1e-12)

    prev = carry_ref[...]
    qs = jnp.concatenate([prev, qn[:-1, :]], axis=0)
    carry_ref[...] = qn[block_rows - 1:block_rows, :]

    cs = jnp.sum(qs * kn, axis=1, keepdims=True)
    temp = jnp.clip(jnp.abs(tb_ref[0]), 0.1, 2.0)
    bias = tb_ref[1]
    p = jax.nn.sigmoid((1.0 - cs + bias) / temp)

    row = jax.lax.broadcasted_iota(jnp.int32, (block_rows, 1), 0)
    gidx = row + i * block_rows
    force = gidx == 0
    for j in range(16):
        force = jnp.logical_or(force, gidx == cu_ref[j])
    p = jnp.where(force, 1.0, p)

    omp = 1.0 - p
    prob_ref[...] = jnp.concatenate([omp, p], axis=1)
    m = p > omp
    mask_ref[...] = m.astype(jnp.float32)
    sel_ref[...] = jnp.where(m, p, omp)


def kernel(hidden_states, cu_seqlens, Wq, Wk, temperature, boundary_bias):
    T, D = hidden_states.shape
    BS = 512
    tb = jnp.stack([temperature.astype(jnp.float32),
                    boundary_bias.astype(jnp.float32)])
    grid_spec = pltpu.PrefetchScalarGridSpec(
        num_scalar_prefetch=2,
        grid=(T // BS,),
        in_specs=[
            pl.BlockSpec((BS, D), lambda i, *_: (i, 0)),
            pl.BlockSpec((D, D), lambda i, *_: (0, 0)),
            pl.BlockSpec((D, D), lambda i, *_: (0, 0)),
        ],
        out_specs=[
            pl.BlockSpec((BS, 2), lambda i, *_: (i, 0)),
            pl.BlockSpec((BS, 1), lambda i, *_: (i, 0)),
            pl.BlockSpec((BS, 1), lambda i, *_: (i, 0)),
        ],
        scratch_shapes=[pltpu.VMEM((1, D), jnp.float32)],
    )
    prob, maskf, sel = pl.pallas_call(
        functools.partial(_routing_body, block_rows=BS),
        grid_spec=grid_spec,
        out_shape=[
            jax.ShapeDtypeStruct((T, 2), jnp.float32),
            jax.ShapeDtypeStruct((T, 1), jnp.float32),
            jax.ShapeDtypeStruct((T, 1), jnp.float32),
        ],
        compiler_params=pltpu.CompilerParams(
            dimension_semantics=("arbitrary",)),
    )(cu_seqlens, tb, hidden_states, Wq, Wk)
    return prob, maskf.reshape(T).astype(bool), sel
